# baseline (device time: 27227 ns/iter reference)
import jax
import jax.numpy as jnp
from jax import lax
from jax.experimental import pallas as pl
from jax.experimental.pallas import tpu as pltpu

XS_Y = (64, 192, 192, 16)
CC = 96
YROWS = sum(XS_Y)
NCH = len(XS_Y)
OFFS = tuple(sum(XS_Y[:j]) for j in range(NCH))
XROWS = YROWS + CC


def kernel(partial, gamma):
    _, m_tot, d = partial.shape
    m_out = m_tot // 2
    cc_rel = 2 * YROWS

    gamma2 = gamma.reshape(1, d)

    def body(p_ref, g_ref, o_ref, mine_ref, ssrc_ref, sx_ref, rx_ref,
             ostage_ref, ry_ref, sem_pf, sem_mine, sem_sx, sem_rx, sem_sy,
             sem_ry, sem_out):
        my_x = lax.axis_index("x")
        my_y = lax.axis_index("y")
        nbr_x = (1 - my_x, my_y)
        nbr_y = (my_x, 1 - my_y)
        mine_y = my_x * m_out + my_y * YROWS
        mine_cc = my_x * m_out + cc_rel
        send_y = (1 - my_x) * m_out + my_y * YROWS
        send_cc = (1 - my_x) * m_out + cc_rel

        pf = []
        for j in range(NCH):
            cp = pltpu.make_async_copy(
                p_ref.at[0, pl.ds(send_y + OFFS[j], XS_Y[j]), :],
                ssrc_ref.at[pl.ds(OFFS[j], XS_Y[j])], sem_pf.at[j])
            cp.start()
            pf.append(cp)
        cp_cc = pltpu.make_async_copy(
            p_ref.at[0, pl.ds(send_cc, CC), :],
            ssrc_ref.at[pl.ds(YROWS, CC)], sem_pf.at[NCH])
        cp_cc.start()
        cp_mine_y = pltpu.make_async_copy(
            p_ref.at[0, pl.ds(mine_y, YROWS), :],
            mine_ref.at[pl.ds(0, YROWS)], sem_mine.at[0])
        cp_mine_y.start()
        cp_mine_cc = pltpu.make_async_copy(
            p_ref.at[0, pl.ds(mine_cc, CC), :],
            mine_ref.at[pl.ds(YROWS, CC)], sem_mine.at[1])
        cp_mine_cc.start()

        bsem = pltpu.get_barrier_semaphore()
        for nbr in (nbr_x, nbr_y):
            pl.semaphore_signal(bsem, inc=1, device_id=nbr,
                                device_id_type=pl.DeviceIdType.MESH)
        pl.semaphore_wait(bsem, 2)

        rdmas_x = []
        for j in range(NCH + 1):
            off, sz = (OFFS[j], XS_Y[j]) if j < NCH else (YROWS, CC)
            (pf[j] if j < NCH else cp_cc).wait()
            sx_ref[pl.ds(off, sz), :] = ssrc_ref[pl.ds(off, sz), :].astype(
                jnp.bfloat16)
            r = pltpu.make_async_remote_copy(
                src_ref=sx_ref.at[pl.ds(off, sz)],
                dst_ref=rx_ref.at[pl.ds(off, sz)],
                send_sem=sem_sx.at[j], recv_sem=sem_rx.at[j],
                device_id=nbr_x, device_id_type=pl.DeviceIdType.MESH)
            r.start()
            rdmas_x.append(r)

        cp_mine_y.wait()
        g = g_ref[0, :]

        def normed(off, sz):
            s = (mine_ref[pl.ds(off, sz), :]
                 + rx_ref[pl.ds(off, sz), :].astype(jnp.float32))
            rms = jnp.sqrt(jnp.mean(s * s, axis=-1, keepdims=True) + 1e-6)
            return (s / rms * g).astype(jnp.bfloat16)

        rdmas_y = []
        out_cps = []
        for j in range(NCH):
            off, sz = OFFS[j], XS_Y[j]
            rdmas_x[j].wait_recv()
            ostage_ref[pl.ds(off, sz), :] = normed(off, sz)
            r = pltpu.make_async_remote_copy(
                src_ref=ostage_ref.at[pl.ds(off, sz)],
                dst_ref=ry_ref.at[pl.ds(off, sz)],
                send_sem=sem_sy.at[j], recv_sem=sem_ry.at[j],
                device_id=nbr_y, device_id_type=pl.DeviceIdType.MESH)
            r.start()
            rdmas_y.append(r)
            cp = pltpu.make_async_copy(
                ostage_ref.at[pl.ds(off, sz)],
                o_ref.at[pl.ds(my_y * YROWS + off, sz)], sem_out.at[j])
            cp.start()
            out_cps.append(cp)

        rdmas_x[NCH].wait_recv()
        cp_mine_cc.wait()
        ostage_ref[pl.ds(YROWS, CC), :] = normed(YROWS, CC)
        cp = pltpu.make_async_copy(
            ostage_ref.at[pl.ds(YROWS, CC)],
            o_ref.at[pl.ds(cc_rel, CC)], sem_out.at[NCH])
        cp.start()
        out_cps.append(cp)

        for j in range(NCH):
            off, sz = OFFS[j], XS_Y[j]
            rdmas_y[j].wait_recv()
            cp = pltpu.make_async_copy(
                ry_ref.at[pl.ds(off, sz)],
                o_ref.at[pl.ds((1 - my_y) * YROWS + off, sz)],
                sem_out.at[NCH + 1 + j])
            cp.start()
            out_cps.append(cp)

        for cp in out_cps:
            cp.wait()
        for r in rdmas_x:
            r.wait_send()
        for r in rdmas_y:
            r.wait_send()

    out_shape = jax.ShapeDtypeStruct((m_out, d), jnp.bfloat16)
    return pl.pallas_call(
        body,
        out_shape=out_shape,
        in_specs=[pl.BlockSpec(memory_space=pl.ANY),
                  pl.BlockSpec(memory_space=pltpu.VMEM)],
        out_specs=pl.BlockSpec(memory_space=pl.ANY),
        scratch_shapes=[
            pltpu.VMEM((XROWS, d), jnp.float32),
            pltpu.VMEM((XROWS, d), jnp.float32),
            pltpu.VMEM((XROWS, d), jnp.bfloat16),
            pltpu.VMEM((XROWS, d), jnp.bfloat16),
            pltpu.VMEM((XROWS, d), jnp.bfloat16),
            pltpu.VMEM((YROWS, d), jnp.bfloat16),
            pltpu.SemaphoreType.DMA((NCH + 1,)),
            pltpu.SemaphoreType.DMA((2,)),
            pltpu.SemaphoreType.DMA((NCH + 1,)),
            pltpu.SemaphoreType.DMA((NCH + 1,)),
            pltpu.SemaphoreType.DMA((NCH,)),
            pltpu.SemaphoreType.DMA((NCH,)),
            pltpu.SemaphoreType.DMA((2 * NCH + 1,)),
        ],
        compiler_params=pltpu.CompilerParams(collective_id=0),
    )(partial, gamma2)
